# baseline (speedup 1.0000x reference)
import jax
import jax.numpy as jnp
from jax.experimental import pallas as pl


def _add_body(emb_ref, x_ref, o_ref):
    o_ref[...] = x_ref[...] + emb_ref[...].reshape(1, 1, 1, 128)


def kernel(x, table):
    B, T, N, C = x.shape
    R = N * C // 128
    x2 = x.reshape(B, T, R, 128)
    table2 = jnp.concatenate([table, table], axis=1).reshape(-1, 1, 128)

    out2 = pl.pallas_call(
        _add_body,
        grid=(B, T),
        in_specs=[
            pl.BlockSpec((1, 1, 128), lambda b, t: (t, 0, 0)),
            pl.BlockSpec((1, 1, R, 128), lambda b, t: (b, t, 0, 0)),
        ],
        out_specs=pl.BlockSpec((1, 1, R, 128), lambda b, t: (b, t, 0, 0)),
        out_shape=jax.ShapeDtypeStruct((B, T, R, 128), x.dtype),
    )(table2, x2)
    return out2.reshape(B, T, N, C)


# speedup vs baseline: 1.2519x; 1.2519x over previous
import jax
import jax.numpy as jnp
from jax.experimental import pallas as pl


def _add_body(emb_ref, x_ref, o_ref):
    o_ref[...] = x_ref[...] + emb_ref[...].reshape(1, 1, 1, 64)


def kernel(x, table):
    B, T, N, C = x.shape
    table3 = table.reshape(-1, 1, C)

    return pl.pallas_call(
        _add_body,
        grid=(B, T),
        in_specs=[
            pl.BlockSpec((1, 1, C), lambda b, t: (t, 0, 0)),
            pl.BlockSpec((1, 1, N, C), lambda b, t: (b, t, 0, 0)),
        ],
        out_specs=pl.BlockSpec((1, 1, N, C), lambda b, t: (b, t, 0, 0)),
        out_shape=jax.ShapeDtypeStruct((B, T, N, C), x.dtype),
    )(table3, x)


# speedup vs baseline: 1.7337x; 1.3848x over previous
import jax
import jax.numpy as jnp
from jax.experimental import pallas as pl

_TBLK = 8


def _add_body(emb_ref, x_ref, o_ref):
    emb = emb_ref[...].reshape(1, _TBLK, 1, 64)
    o_ref[...] = x_ref[...] + emb


def kernel(x, table):
    B, T, N, C = x.shape
    return pl.pallas_call(
        _add_body,
        grid=(B, T // _TBLK),
        in_specs=[
            pl.BlockSpec((_TBLK, C), lambda b, t: (t, 0)),
            pl.BlockSpec((1, _TBLK, N, C), lambda b, t: (b, t, 0, 0)),
        ],
        out_specs=pl.BlockSpec((1, _TBLK, N, C), lambda b, t: (b, t, 0, 0)),
        out_shape=jax.ShapeDtypeStruct((B, T, N, C), x.dtype),
    )(table, x)
